# baseline (speedup 1.0000x reference)
import functools

import jax
import jax.numpy as jnp
from jax import lax
from jax.experimental import pallas as pl
from jax.experimental.pallas import tpu as pltpu
from jax.experimental.pallas import tpu_sc as plsc

_NUM_SCALES = 16
_DIM = 64
_LANES = 16
_GROUPS = _DIM // _LANES

_NC = 2
_NS = 16
_NW = _NC * _NS

_CHUNK = 512


_NBUF = 3


def _sc_body(x_hbm, p_hbm, emb_hbm, out_hbm,
             buf0, buf1, buf2, idx0, idx1, idx2, table,
             sin0, sin1, sin2, sout0, sout1, sout2):
    bufs = (buf0, buf1, buf2)
    idxs = (idx0, idx1, idx2)
    sins = (sin0, sin1, sin2)
    souts = (sout0, sout1, sout2)

    n_rows = x_hbm.shape[0] // _DIM
    rows_per_w = n_rows // _NW
    n_chunks = rows_per_w // _CHUNK

    wid = lax.axis_index("s") * _NC + lax.axis_index("c")
    w_base = wid * rows_per_w

    def start_in(g, b):
        start = w_base + g * _CHUNK
        pltpu.async_copy(
            x_hbm.at[pl.ds(start * _DIM, _CHUNK * _DIM)], bufs[b], sins[b])
        pltpu.async_copy(p_hbm.at[pl.ds(start, _CHUNK)], idxs[b], sins[b])

    def wait_in(b):
        pltpu.make_async_copy(
            x_hbm.at[pl.ds(0, _CHUNK * _DIM)], bufs[b], sins[b]).wait()
        pltpu.make_async_copy(
            p_hbm.at[pl.ds(0, _CHUNK)], idxs[b], sins[b]).wait()

    def start_out(g, b):
        start = w_base + g * _CHUNK
        pltpu.async_copy(
            bufs[b], out_hbm.at[pl.ds(start * _DIM, _CHUNK * _DIM)], souts[b])

    def wait_out(b):
        pltpu.make_async_copy(
            bufs[b], out_hbm.at[pl.ds(0, _CHUNK * _DIM)], souts[b]).wait()

    def compute(b):
        buf, idxbuf = bufs[b], idxs[b]

        @plsc.parallel_loop(0, _CHUNK // _LANES, unroll=1)
        def row_body(rb):
            r0 = rb * _LANES
            pvec = idxbuf[pl.ds(r0, _LANES)]
            for i in range(_LANES):
                rbase = (r0 + i) * _DIM
                ebase = pvec[i] * _DIM
                ins = [buf[pl.ds(rbase + q * _LANES, _LANES)]
                       for q in range(_GROUPS)]
                embs = [table[pl.ds(ebase + q * _LANES, _LANES)]
                        for q in range(_GROUPS)]
                sums = [a + c for a, c in zip(ins, embs)]
                for q in range(_GROUPS):
                    buf[pl.ds(rbase + q * _LANES, _LANES)] = sums[q]

    pltpu.sync_copy(emb_hbm, table)

    start_in(0, 0)
    start_in(1, 1)

    def outer(go, carry):
        for b in range(_NBUF):
            g = go * _NBUF + b
            wait_in(b)
            compute(b)
            start_out(g, b)
            zb = (b + 2) % _NBUF
            if b == 0:
                @pl.when(go > 0)
                def _():
                    wait_out(zb)
            else:
                wait_out(zb)
            start_in(g + 2, zb)
        return carry

    lax.fori_loop(0, (n_chunks - 2) // _NBUF, outer, 0)

    for g, b in ((n_chunks - 2, (n_chunks - 2) % _NBUF),
                 (n_chunks - 1, (n_chunks - 1) % _NBUF)):
        wait_in(b)
        compute(b)
        start_out(g, b)
    for b in range(_NBUF):
        wait_out(b)


def kernel(inputs, inputs_scale_positions, scale_emb):
    b, l, d = inputs.shape
    n = b * l
    x = inputs.reshape(n * d)
    p = inputs_scale_positions.reshape(n)
    emb = scale_emb.reshape(_NUM_SCALES * d)

    mesh = plsc.VectorSubcoreMesh(core_axis_name="c", subcore_axis_name="s")
    run = pl.kernel(
        _sc_body,
        mesh=mesh,
        out_type=jax.ShapeDtypeStruct((n * d,), jnp.float32),
        scratch_types=(
            [pltpu.VMEM((_CHUNK * d,), jnp.float32) for _ in range(_NBUF)]
            + [pltpu.VMEM((_CHUNK,), jnp.int32) for _ in range(_NBUF)]
            + [pltpu.VMEM((_NUM_SCALES * d,), jnp.float32)]
            + [pltpu.SemaphoreType.DMA for _ in range(2 * _NBUF)]
        ),
    )
    out = run(x, p, emb)
    return out.reshape(b, l, d)


# speedup vs baseline: 1.7302x; 1.7302x over previous
import functools

import jax
import jax.numpy as jnp
from jax import lax
from jax.experimental import pallas as pl
from jax.experimental.pallas import tpu as pltpu
from jax.experimental.pallas import tpu_sc as plsc

_NUM_SCALES = 16
_DIM = 64
_LANES = 16
_GROUPS = _DIM // _LANES

_NC = 2
_NS = 16
_NW = _NC * _NS

_CHUNK = 320


_NBUF = 3


def _sc_body(x_hbm, p_hbm, emb_hbm, out_hbm,
             buf0, buf1, buf2, idx0, idx1, idx2, table,
             sin0, sin1, sin2, sout0, sout1, sout2):
    bufs = (buf0, buf1, buf2)
    idxs = (idx0, idx1, idx2)
    sins = (sin0, sin1, sin2)
    souts = (sout0, sout1, sout2)

    n_rows = x_hbm.shape[0]
    rows_per_w = n_rows // _NW
    n_chunks = rows_per_w // _CHUNK

    wid = lax.axis_index("s") * _NC + lax.axis_index("c")
    w_base = wid * rows_per_w

    def start_in(g, b):
        start = w_base + g * _CHUNK
        pltpu.async_copy(x_hbm.at[pl.ds(start, _CHUNK)], bufs[b], sins[b])
        pltpu.async_copy(p_hbm.at[pl.ds(start, _CHUNK)], idxs[b], sins[b])

    def wait_in(b):
        pltpu.make_async_copy(
            x_hbm.at[pl.ds(0, _CHUNK)], bufs[b], sins[b]).wait()
        pltpu.make_async_copy(
            p_hbm.at[pl.ds(0, _CHUNK)], idxs[b], sins[b]).wait()

    def start_out(g, b):
        start = w_base + g * _CHUNK
        pltpu.async_copy(bufs[b], out_hbm.at[pl.ds(start, _CHUNK)], souts[b])

    def wait_out(b):
        pltpu.make_async_copy(
            bufs[b], out_hbm.at[pl.ds(0, _CHUNK)], souts[b]).wait()

    def compute(b):
        buf, idxbuf = bufs[b], idxs[b]

        @plsc.parallel_loop(0, _CHUNK // _LANES, unroll=1)
        def row_body(rb):
            r0 = rb * _LANES
            pvec = idxbuf[pl.ds(r0, _LANES)]
            for i in range(_LANES):
                p = pvec[i]
                ins = [buf[r0 + i, pl.ds(q * _LANES, _LANES)]
                       for q in range(_GROUPS)]
                embs = [table[p, pl.ds(q * _LANES, _LANES)]
                        for q in range(_GROUPS)]
                sums = [a + c for a, c in zip(ins, embs)]
                for q in range(_GROUPS):
                    buf[r0 + i, pl.ds(q * _LANES, _LANES)] = sums[q]

    pltpu.sync_copy(emb_hbm, table)

    start_in(0, 0)
    start_in(1, 1)

    def outer(go, carry):
        for b in range(_NBUF):
            g = go * _NBUF + b
            wait_in(b)
            compute(b)
            start_out(g, b)
            zb = (b + 2) % _NBUF
            if b == 0:
                @pl.when(go > 0)
                def _():
                    wait_out(zb)
            else:
                wait_out(zb)
            start_in(g + 2, zb)
        return carry

    lax.fori_loop(0, (n_chunks - 2) // _NBUF, outer, 0)

    for g, b in ((n_chunks - 2, (n_chunks - 2) % _NBUF),
                 (n_chunks - 1, (n_chunks - 1) % _NBUF)):
        wait_in(b)
        compute(b)
        start_out(g, b)
    for b in range(_NBUF):
        wait_out(b)


def kernel(inputs, inputs_scale_positions, scale_emb):
    b, l, d = inputs.shape
    n = b * l
    x = inputs.reshape(n, d)
    p = inputs_scale_positions.reshape(n)

    mesh = plsc.VectorSubcoreMesh(core_axis_name="c", subcore_axis_name="s")
    run = pl.kernel(
        _sc_body,
        mesh=mesh,
        compiler_params=pltpu.CompilerParams(use_tc_tiling_on_sc=True),
        out_type=jax.ShapeDtypeStruct((n, d), jnp.float32),
        scratch_types=(
            [pltpu.VMEM((_CHUNK, d), jnp.float32) for _ in range(_NBUF)]
            + [pltpu.VMEM((_CHUNK,), jnp.int32) for _ in range(_NBUF)]
            + [pltpu.VMEM((_NUM_SCALES, d), jnp.float32)]
            + [pltpu.SemaphoreType.DMA for _ in range(2 * _NBUF)]
        ),
    )
    out = run(x, p, scale_emb)
    return out.reshape(b, l, d)
